# baseline (speedup 1.0000x reference)
import jax
import jax.numpy as jnp
from jax import lax
from jax.experimental import pallas as pl
from jax.experimental.pallas import tpu as pltpu
from jax.experimental.pallas import tpu_sc as plsc

N_NODES = 10000
D_FEAT = 128
N_EDGES = 320000
M_ITEMS = 1000
EMB = 64
B = 4096

NC = 2
NS = 16
NW = NC * NS
CH = 128
EW = N_EDGES // NW
NCH = 79
E_PAD = NW * NCH * CH
N_ACC = N_NODES + 112
ROWS_PER_TILE = N_ACC // NS


def _f32(x):
    return x.astype(jnp.float32)


def _make_edge_agg(with_cnt: bool):
    mesh = plsc.VectorSubcoreMesh(core_axis_name="c", subcore_axis_name="s")
    out_type = [jax.ShapeDtypeStruct((NC, N_ACC, EMB), jnp.float32)]
    scratch = [
        pltpu.VMEM((NCH, CH), jnp.int32),
        pltpu.VMEM((NCH, CH), jnp.int32),
        pltpu.VMEM((CH, EMB), jnp.float32),
        pltpu.VMEM((CH, EMB), jnp.float32),
        pltpu.VMEM_SHARED((N_ACC, EMB), jnp.float32),
        pltpu.SemaphoreType.DMA,
        pltpu.SemaphoreType.DMA,
    ]
    if with_cnt:
        out_type.append(jax.ShapeDtypeStruct((NC, N_ACC, 16), jnp.float32))
        scratch += [
            pltpu.VMEM((CH, 16), jnp.float32),
            pltpu.VMEM_SHARED((N_ACC, 16), jnp.float32),
        ]

    def body(p_hbm, src_hbm, dst_hbm, z64_hbm, z16_hbm, ones_hbm, *refs):
        if with_cnt:
            out_hbm, cnt_hbm = refs[0], refs[1]
            refs = refs[2:]
        else:
            out_hbm = refs[0]
            refs = refs[1:]
        (src_v, dst_v, rows_a, rows_b, acc_s, gsem_a, gsem_b) = refs[:7]
        if with_cnt:
            ones_v, cnt_s = refs[7], refs[8]
        c = lax.axis_index("c")
        s = lax.axis_index("s")
        wid = s * NC + c
        r0 = s * ROWS_PER_TILE
        pltpu.sync_copy(z64_hbm.at[pl.ds(r0, ROWS_PER_TILE)],
                        acc_s.at[pl.ds(r0, ROWS_PER_TILE)])
        if with_cnt:
            pltpu.sync_copy(z16_hbm.at[pl.ds(r0, ROWS_PER_TILE)],
                            cnt_s.at[pl.ds(r0, ROWS_PER_TILE)])
            pltpu.sync_copy(ones_hbm, ones_v)
        pltpu.sync_copy(src_hbm.at[wid], src_v)
        pltpu.sync_copy(dst_hbm.at[wid], dst_v)
        plsc.subcore_barrier()

        def gather(j, rows, sem):
            pltpu.async_copy(p_hbm.at[src_v.at[j]], rows, sem)

        def gwait(j, rows, sem):
            pltpu.make_async_copy(p_hbm.at[src_v.at[j]], rows, sem).wait()

        def scatter(j, rows):
            pltpu.sync_copy(rows, acc_s.at[dst_v.at[j]], add=True)
            if with_cnt:
                pltpu.sync_copy(ones_v, cnt_s.at[dst_v.at[j]], add=True)

        gather(0, rows_a, gsem_a)

        def step(k, carry):
            ja = 2 * k
            jb = 2 * k + 1

            @pl.when(jb < NCH)
            def _():
                gather(jb, rows_b, gsem_b)

            gwait(ja, rows_a, gsem_a)
            scatter(ja, rows_a)

            @pl.when(ja + 2 < NCH)
            def _():
                gather(ja + 2, rows_a, gsem_a)

            @pl.when(jb < NCH)
            def _():
                gwait(jb, rows_b, gsem_b)
                scatter(jb, rows_b)

            return carry

        lax.fori_loop(0, (NCH + 1) // 2, step, 0)
        plsc.subcore_barrier()
        pltpu.sync_copy(acc_s.at[pl.ds(r0, ROWS_PER_TILE)],
                        out_hbm.at[c, pl.ds(r0, ROWS_PER_TILE)])
        if with_cnt:
            pltpu.sync_copy(cnt_s.at[pl.ds(r0, ROWS_PER_TILE)],
                            cnt_hbm.at[c, pl.ds(r0, ROWS_PER_TILE)])

    return pl.kernel(body, out_type=tuple(out_type), mesh=mesh,
                     scratch_types=scratch,
                     compiler_params=pltpu.CompilerParams(
                         use_tc_tiling_on_sc=False))


_edge_agg_cnt = _make_edge_agg(True)
_edge_agg = _make_edge_agg(False)


_BG = B // NW


def _gather4_body(a2, t2, t3, x_hbm, o0, o1, o2, o3,
                  idx_v, b0, b1, b2, b3, sem):
    t0 = a2.at[0]
    t1 = a2.at[1]
    c = lax.axis_index("c")
    s = lax.axis_index("s")
    base = (s * NC + c) * _BG
    pltpu.sync_copy(x_hbm.at[pl.ds(base, _BG)], idx_v)
    pltpu.async_copy(t0.at[idx_v], b0, sem)
    pltpu.async_copy(t1.at[idx_v], b1, sem)
    pltpu.async_copy(t2.at[idx_v], b2, sem)
    pltpu.async_copy(t3.at[idx_v], b3, sem)
    pltpu.make_async_copy(t0.at[idx_v], b0, sem).wait()
    pltpu.make_async_copy(t1.at[idx_v], b1, sem).wait()
    pltpu.make_async_copy(t2.at[idx_v], b2, sem).wait()
    pltpu.make_async_copy(t3.at[idx_v], b3, sem).wait()
    pltpu.sync_copy(b0, o0.at[pl.ds(base, _BG)])
    pltpu.sync_copy(b1, o1.at[pl.ds(base, _BG)])
    pltpu.sync_copy(b2, o2.at[pl.ds(base, _BG)])
    pltpu.sync_copy(b3, o3.at[pl.ds(base, _BG)])


_gather4 = pl.kernel(
    _gather4_body,
    out_type=(
        jax.ShapeDtypeStruct((B, EMB), jnp.float32),
        jax.ShapeDtypeStruct((B, EMB), jnp.float32),
        jax.ShapeDtypeStruct((B, EMB), jnp.float32),
        jax.ShapeDtypeStruct((B, 16), jnp.float32),
    ),
    mesh=plsc.VectorSubcoreMesh(core_axis_name="c", subcore_axis_name="s"),
    scratch_types=[
        pltpu.VMEM((_BG,), jnp.int32),
        pltpu.VMEM((_BG, EMB), jnp.float32),
        pltpu.VMEM((_BG, EMB), jnp.float32),
        pltpu.VMEM((_BG, EMB), jnp.float32),
        pltpu.VMEM((_BG, 16), jnp.float32),
        pltpu.SemaphoreType.DMA,
    ],
    compiler_params=pltpu.CompilerParams(use_tc_tiling_on_sc=False),
)


_RB = 1000


def _dgT(x, w):
    return lax.dot_general(x, w, (((1,), (1,)), ((), ())),
                           preferred_element_type=jnp.float32)


def _pre_body(nx_ref, rat_ref, w1l_ref, w1r_ref, wenc_ref,
              p1_ref, xr_ref, renc_ref):
    nx = nx_ref[...]
    p1_ref[...] = _dgT(nx, w1l_ref[...])
    xr_ref[...] = _dgT(nx, w1r_ref[...])
    renc_ref[...] = _dgT(rat_ref[...], wenc_ref[...])


def _tc_pre(node_x, rating_mat, w1l, w1r, wenc):
    n = node_x.shape[0]
    grid = (n // _RB,)
    return pl.pallas_call(
        _pre_body,
        grid=grid,
        in_specs=[
            pl.BlockSpec((_RB, D_FEAT), lambda i: (i, 0)),
            pl.BlockSpec((_RB, M_ITEMS), lambda i: (i, 0)),
            pl.BlockSpec((EMB, D_FEAT), lambda i: (0, 0)),
            pl.BlockSpec((EMB, D_FEAT), lambda i: (0, 0)),
            pl.BlockSpec((EMB, M_ITEMS), lambda i: (0, 0)),
        ],
        out_specs=[
            pl.BlockSpec((_RB, EMB), lambda i: (i, 0)),
            pl.BlockSpec((_RB, EMB), lambda i: (i, 0)),
            pl.BlockSpec((_RB, EMB), lambda i: (i, 0)),
        ],
        out_shape=[
            jax.ShapeDtypeStruct((n, EMB), jnp.float32),
            jax.ShapeDtypeStruct((n, EMB), jnp.float32),
            jax.ShapeDtypeStruct((n, EMB), jnp.float32),
        ],
    )(node_x, rating_mat, w1l, w1r, wenc)


def _mid_body(agg_ref, cnt_ref, xr_ref, mask2_ref, renc_ref, b1l_ref,
              bias_ref, w2l_ref, w2r_ref, p2_ref, hrb_ref, csum_ref):
    a = agg_ref[0] + agg_ref[1]
    csum = cnt_ref[0] + cnt_ref[1]
    mean = a / jnp.maximum(csum[:, 0:1], 1.0)
    h = jnp.maximum(mean + b1l_ref[...] + xr_ref[...], 0.0) * mask2_ref[...]
    p2_ref[...] = _dgT(h, w2l_ref[...])
    hrb_ref[...] = (_dgT(h, w2r_ref[...]) + renc_ref[...] + bias_ref[...])
    csum_ref[...] = csum


def _tc_mid(agg1, cnt, xr, mask2, renc, b1l, bias, w2l, w2r):
    n = xr.shape[0]
    grid = (n // _RB,)
    return pl.pallas_call(
        _mid_body,
        grid=grid,
        in_specs=[
            pl.BlockSpec((NC, _RB, EMB), lambda i: (0, i, 0)),
            pl.BlockSpec((NC, _RB, 16), lambda i: (0, i, 0)),
            pl.BlockSpec((_RB, EMB), lambda i: (i, 0)),
            pl.BlockSpec((_RB, EMB), lambda i: (i, 0)),
            pl.BlockSpec((_RB, EMB), lambda i: (i, 0)),
            pl.BlockSpec((1, EMB), lambda i: (0, 0)),
            pl.BlockSpec((1, EMB), lambda i: (0, 0)),
            pl.BlockSpec((EMB, EMB), lambda i: (0, 0)),
            pl.BlockSpec((EMB, EMB), lambda i: (0, 0)),
        ],
        out_specs=[
            pl.BlockSpec((_RB, EMB), lambda i: (i, 0)),
            pl.BlockSpec((_RB, EMB), lambda i: (i, 0)),
            pl.BlockSpec((_RB, 16), lambda i: (i, 0)),
        ],
        out_shape=[
            jax.ShapeDtypeStruct((n, EMB), jnp.float32),
            jax.ShapeDtypeStruct((n, EMB), jnp.float32),
            jax.ShapeDtypeStruct((n, 16), jnp.float32),
        ],
    )(agg1, cnt, xr, mask2, renc, b1l, bias, w2l, w2r)


_DB = 512


def _dec_body(a0_ref, a1_ref, hx_ref, cx_ref, wdec_ref, bdec_ref, out_ref):
    mean = ((a0_ref[...] + a1_ref[...])
            / jnp.maximum(cx_ref[:, 0:1], 1.0))
    t = jax.nn.sigmoid(mean + hx_ref[...])
    y = _dgT(t, wdec_ref[...])
    out_ref[...] = jax.nn.sigmoid(y + bdec_ref[...])


def _tc_dec(a0x, a1x, hx, cx, wdec, bdec):
    grid = (B // _DB,)
    return pl.pallas_call(
        _dec_body,
        grid=grid,
        in_specs=[
            pl.BlockSpec((_DB, EMB), lambda i: (i, 0)),
            pl.BlockSpec((_DB, EMB), lambda i: (i, 0)),
            pl.BlockSpec((_DB, EMB), lambda i: (i, 0)),
            pl.BlockSpec((_DB, 16), lambda i: (i, 0)),
            pl.BlockSpec((M_ITEMS, EMB), lambda i: (0, 0)),
            pl.BlockSpec((1, M_ITEMS), lambda i: (0, 0)),
        ],
        out_specs=pl.BlockSpec((_DB, M_ITEMS), lambda i: (i, 0)),
        out_shape=jax.ShapeDtypeStruct((B, M_ITEMS), jnp.float32),
    )(a0x, a1x, hx, cx, wdec, bdec)


def kernel(x, rating_mat, node_x, edge_index, user_table,
           w1l, b1l, w1r, w2l, b2l, w2r,
           w_enc, b_enc, w_dec, b_dec):
    del user_table
    x = x.astype(jnp.int32)
    src = edge_index[0].astype(jnp.int32)
    dst = edge_index[1].astype(jnp.int32)
    pad = E_PAD - N_EDGES
    pad_dst = N_NODES + jnp.arange(pad, dtype=jnp.int32) % (N_ACC - N_NODES)
    src_p = jnp.concatenate([src, jnp.zeros((pad,), jnp.int32)])
    dst_p = jnp.concatenate([dst, pad_dst])
    src_p = src_p.reshape(NW, NCH, CH)
    dst_p = dst_p.reshape(NW, NCH, CH)
    z64 = jnp.zeros((N_ACC, EMB), jnp.float32)
    z16 = jnp.zeros((N_ACC, 16), jnp.float32)
    ones = jnp.ones((CH, 16), jnp.float32)

    keep = jax.random.bernoulli(jax.random.key(42), 0.5, (N_NODES, EMB))
    mask2 = keep.astype(jnp.float32) * 2.0

    p1, xr, renc = _tc_pre(node_x, rating_mat, w1l, w1r, w_enc)
    agg1, cnt = _edge_agg_cnt(p1, src_p, dst_p, z64, z16, ones)
    bias = (b2l + b_enc).reshape(1, EMB)
    p2, hrb, csum = _tc_mid(agg1, cnt, xr, mask2, renc,
                            b1l.reshape(1, EMB), bias, w2l, w2r)
    (agg2,) = _edge_agg(p2, src_p, dst_p, z64, z16, ones)
    a0x, a1x, hx, cx = _gather4(agg2, hrb, csum, x)
    return _tc_dec(a0x, a1x, hx, cx, w_dec, b_dec.reshape(1, M_ITEMS))


# speedup vs baseline: 1.1085x; 1.1085x over previous
import jax
import jax.numpy as jnp
from jax import lax
from jax.experimental import pallas as pl
from jax.experimental.pallas import tpu as pltpu
from jax.experimental.pallas import tpu_sc as plsc

N_NODES = 10000
D_FEAT = 128
N_EDGES = 320000
M_ITEMS = 1000
EMB = 64
B = 4096

NC = 2
NS = 16
NW = NC * NS
CH = 128
EW = N_EDGES // NW
NCH_F = 100
NCH_S = 58
NCH = NCH_F
E_PAD = NS * (NCH_F + NCH_S) * CH
N_ACC = N_NODES + 112
ROWS_PER_TILE = N_ACC // NS


def _f32(x):
    return x.astype(jnp.float32)


def _make_edge_agg(with_cnt: bool):
    mesh = plsc.VectorSubcoreMesh(core_axis_name="c", subcore_axis_name="s")
    out_type = [jax.ShapeDtypeStruct((NC, N_ACC, EMB), jnp.float32)]
    scratch = [
        pltpu.VMEM((NCH, CH), jnp.int32),
        pltpu.VMEM((NCH, CH), jnp.int32),
        pltpu.VMEM((CH, EMB), jnp.float32),
        pltpu.VMEM((CH, EMB), jnp.float32),
        pltpu.VMEM_SHARED((N_ACC, EMB), jnp.float32),
        pltpu.SemaphoreType.DMA,
        pltpu.SemaphoreType.DMA,
    ]
    if with_cnt:
        out_type.append(jax.ShapeDtypeStruct((NC, N_ACC, 16), jnp.float32))
        scratch += [
            pltpu.VMEM((CH, 16), jnp.float32),
            pltpu.VMEM_SHARED((N_ACC, 16), jnp.float32),
        ]

    def body(p_hbm, src_hbm, dst_hbm, z64_hbm, z16_hbm, ones_hbm, *refs):
        if with_cnt:
            out_hbm, cnt_hbm = refs[0], refs[1]
            refs = refs[2:]
        else:
            out_hbm = refs[0]
            refs = refs[1:]
        (src_v, dst_v, rows_a, rows_b, acc_s, gsem_a, gsem_b) = refs[:7]
        if with_cnt:
            ones_v, cnt_s = refs[7], refs[8]
        c = lax.axis_index("c")
        s = lax.axis_index("s")
        wid = s * NC + c
        nch = jnp.where(c == 0, NCH_F, NCH_S)
        r0 = s * ROWS_PER_TILE
        pltpu.sync_copy(z64_hbm.at[pl.ds(r0, ROWS_PER_TILE)],
                        acc_s.at[pl.ds(r0, ROWS_PER_TILE)])
        if with_cnt:
            pltpu.sync_copy(z16_hbm.at[pl.ds(r0, ROWS_PER_TILE)],
                            cnt_s.at[pl.ds(r0, ROWS_PER_TILE)])
            pltpu.sync_copy(ones_hbm, ones_v)
        pltpu.sync_copy(src_hbm.at[wid], src_v)
        pltpu.sync_copy(dst_hbm.at[wid], dst_v)
        plsc.subcore_barrier()

        def gather(j, rows, sem):
            pltpu.async_copy(p_hbm.at[src_v.at[j]], rows, sem)

        def gwait(j, rows, sem):
            pltpu.make_async_copy(p_hbm.at[src_v.at[j]], rows, sem).wait()

        def scatter(j, rows):
            pltpu.sync_copy(rows, acc_s.at[dst_v.at[j]], add=True)
            if with_cnt:
                pltpu.sync_copy(ones_v, cnt_s.at[dst_v.at[j]], add=True)

        gather(0, rows_a, gsem_a)

        def step(k, carry):
            ja = 2 * k
            jb = 2 * k + 1
            gather(jb, rows_b, gsem_b)
            gwait(ja, rows_a, gsem_a)
            scatter(ja, rows_a)

            @pl.when(ja + 2 < nch)
            def _():
                gather(ja + 2, rows_a, gsem_a)

            gwait(jb, rows_b, gsem_b)
            scatter(jb, rows_b)
            return carry

        lax.fori_loop(0, nch // 2, step, 0)
        plsc.subcore_barrier()
        pltpu.sync_copy(acc_s.at[pl.ds(r0, ROWS_PER_TILE)],
                        out_hbm.at[c, pl.ds(r0, ROWS_PER_TILE)])
        if with_cnt:
            pltpu.sync_copy(cnt_s.at[pl.ds(r0, ROWS_PER_TILE)],
                            cnt_hbm.at[c, pl.ds(r0, ROWS_PER_TILE)])

    return pl.kernel(body, out_type=tuple(out_type), mesh=mesh,
                     scratch_types=scratch,
                     compiler_params=pltpu.CompilerParams(
                         use_tc_tiling_on_sc=False))


_edge_agg_cnt = _make_edge_agg(True)
_edge_agg = _make_edge_agg(False)


_BG = B // NW


def _gather4_body(a2, t2, t3, x_hbm, o0, o1, o2, o3,
                  idx_v, b0, b1, b2, b3, sem):
    t0 = a2.at[0]
    t1 = a2.at[1]
    c = lax.axis_index("c")
    s = lax.axis_index("s")
    base = (s * NC + c) * _BG
    pltpu.sync_copy(x_hbm.at[pl.ds(base, _BG)], idx_v)
    pltpu.async_copy(t0.at[idx_v], b0, sem)
    pltpu.async_copy(t1.at[idx_v], b1, sem)
    pltpu.async_copy(t2.at[idx_v], b2, sem)
    pltpu.async_copy(t3.at[idx_v], b3, sem)
    pltpu.make_async_copy(t0.at[idx_v], b0, sem).wait()
    pltpu.make_async_copy(t1.at[idx_v], b1, sem).wait()
    pltpu.make_async_copy(t2.at[idx_v], b2, sem).wait()
    pltpu.make_async_copy(t3.at[idx_v], b3, sem).wait()
    pltpu.sync_copy(b0, o0.at[pl.ds(base, _BG)])
    pltpu.sync_copy(b1, o1.at[pl.ds(base, _BG)])
    pltpu.sync_copy(b2, o2.at[pl.ds(base, _BG)])
    pltpu.sync_copy(b3, o3.at[pl.ds(base, _BG)])


_gather4 = pl.kernel(
    _gather4_body,
    out_type=(
        jax.ShapeDtypeStruct((B, EMB), jnp.float32),
        jax.ShapeDtypeStruct((B, EMB), jnp.float32),
        jax.ShapeDtypeStruct((B, EMB), jnp.float32),
        jax.ShapeDtypeStruct((B, 16), jnp.float32),
    ),
    mesh=plsc.VectorSubcoreMesh(core_axis_name="c", subcore_axis_name="s"),
    scratch_types=[
        pltpu.VMEM((_BG,), jnp.int32),
        pltpu.VMEM((_BG, EMB), jnp.float32),
        pltpu.VMEM((_BG, EMB), jnp.float32),
        pltpu.VMEM((_BG, EMB), jnp.float32),
        pltpu.VMEM((_BG, 16), jnp.float32),
        pltpu.SemaphoreType.DMA,
    ],
    compiler_params=pltpu.CompilerParams(use_tc_tiling_on_sc=False),
)


_RB = 1000


def _dgT(x, w):
    return lax.dot_general(x, w, (((1,), (1,)), ((), ())),
                           preferred_element_type=jnp.float32)


def _pre_body(nx_ref, w1l_ref, w1r_ref, p1_ref, xr_ref):
    nx = nx_ref[...]
    p1_ref[...] = _dgT(nx, w1l_ref[...])
    xr_ref[...] = _dgT(nx, w1r_ref[...])


def _tc_pre(node_x, w1l, w1r):
    n = node_x.shape[0]
    grid = (n // _RB,)
    return pl.pallas_call(
        _pre_body,
        grid=grid,
        in_specs=[
            pl.BlockSpec((_RB, D_FEAT), lambda i: (i, 0)),
            pl.BlockSpec((EMB, D_FEAT), lambda i: (0, 0)),
            pl.BlockSpec((EMB, D_FEAT), lambda i: (0, 0)),
        ],
        out_specs=[
            pl.BlockSpec((_RB, EMB), lambda i: (i, 0)),
            pl.BlockSpec((_RB, EMB), lambda i: (i, 0)),
        ],
        out_shape=[
            jax.ShapeDtypeStruct((n, EMB), jnp.float32),
            jax.ShapeDtypeStruct((n, EMB), jnp.float32),
        ],
    )(node_x, w1l, w1r)


def _renc_body(rat_ref, wenc_ref, renc_ref):
    renc_ref[...] = _dgT(rat_ref[...], wenc_ref[...])


def _tc_renc(rating_mat, wenc):
    n = rating_mat.shape[0]
    grid = (n // _RB,)
    return pl.pallas_call(
        _renc_body,
        grid=grid,
        in_specs=[
            pl.BlockSpec((_RB, M_ITEMS), lambda i: (i, 0)),
            pl.BlockSpec((EMB, M_ITEMS), lambda i: (0, 0)),
        ],
        out_specs=pl.BlockSpec((_RB, EMB), lambda i: (i, 0)),
        out_shape=jax.ShapeDtypeStruct((n, EMB), jnp.float32),
    )(rating_mat, wenc)


def _mid_body(agg_ref, cnt_ref, xr_ref, mask2_ref, renc_ref, b1l_ref,
              bias_ref, w2l_ref, w2r_ref, p2_ref, hrb_ref, csum_ref):
    a = agg_ref[0] + agg_ref[1]
    csum = cnt_ref[0] + cnt_ref[1]
    mean = a / jnp.maximum(csum[:, 0:1], 1.0)
    h = jnp.maximum(mean + b1l_ref[...] + xr_ref[...], 0.0) * mask2_ref[...]
    p2_ref[...] = _dgT(h, w2l_ref[...])
    hrb_ref[...] = (_dgT(h, w2r_ref[...]) + renc_ref[...] + bias_ref[...])
    csum_ref[...] = csum


def _tc_mid(agg1, cnt, xr, mask2, renc, b1l, bias, w2l, w2r):
    n = xr.shape[0]
    grid = (n // _RB,)
    return pl.pallas_call(
        _mid_body,
        grid=grid,
        in_specs=[
            pl.BlockSpec((NC, _RB, EMB), lambda i: (0, i, 0)),
            pl.BlockSpec((NC, _RB, 16), lambda i: (0, i, 0)),
            pl.BlockSpec((_RB, EMB), lambda i: (i, 0)),
            pl.BlockSpec((_RB, EMB), lambda i: (i, 0)),
            pl.BlockSpec((_RB, EMB), lambda i: (i, 0)),
            pl.BlockSpec((1, EMB), lambda i: (0, 0)),
            pl.BlockSpec((1, EMB), lambda i: (0, 0)),
            pl.BlockSpec((EMB, EMB), lambda i: (0, 0)),
            pl.BlockSpec((EMB, EMB), lambda i: (0, 0)),
        ],
        out_specs=[
            pl.BlockSpec((_RB, EMB), lambda i: (i, 0)),
            pl.BlockSpec((_RB, EMB), lambda i: (i, 0)),
            pl.BlockSpec((_RB, 16), lambda i: (i, 0)),
        ],
        out_shape=[
            jax.ShapeDtypeStruct((n, EMB), jnp.float32),
            jax.ShapeDtypeStruct((n, EMB), jnp.float32),
            jax.ShapeDtypeStruct((n, 16), jnp.float32),
        ],
    )(agg1, cnt, xr, mask2, renc, b1l, bias, w2l, w2r)


_DB = 512


def _dec_body(a0_ref, a1_ref, hx_ref, cx_ref, wdec_ref, bdec_ref, out_ref):
    mean = ((a0_ref[...] + a1_ref[...])
            / jnp.maximum(cx_ref[:, 0:1], 1.0))
    t = jax.nn.sigmoid(mean + hx_ref[...])
    y = _dgT(t, wdec_ref[...])
    out_ref[...] = jax.nn.sigmoid(y + bdec_ref[...])


def _tc_dec(a0x, a1x, hx, cx, wdec, bdec):
    grid = (B // _DB,)
    return pl.pallas_call(
        _dec_body,
        grid=grid,
        in_specs=[
            pl.BlockSpec((_DB, EMB), lambda i: (i, 0)),
            pl.BlockSpec((_DB, EMB), lambda i: (i, 0)),
            pl.BlockSpec((_DB, EMB), lambda i: (i, 0)),
            pl.BlockSpec((_DB, 16), lambda i: (i, 0)),
            pl.BlockSpec((M_ITEMS, EMB), lambda i: (0, 0)),
            pl.BlockSpec((1, M_ITEMS), lambda i: (0, 0)),
        ],
        out_specs=pl.BlockSpec((_DB, M_ITEMS), lambda i: (i, 0)),
        out_shape=jax.ShapeDtypeStruct((B, M_ITEMS), jnp.float32),
    )(a0x, a1x, hx, cx, wdec, bdec)


def kernel(x, rating_mat, node_x, edge_index, user_table,
           w1l, b1l, w1r, w2l, b2l, w2r,
           w_enc, b_enc, w_dec, b_dec):
    del user_table
    x = x.astype(jnp.int32)
    src = edge_index[0].astype(jnp.int32)
    dst = edge_index[1].astype(jnp.int32)
    pad = E_PAD - N_EDGES
    pad_dst = N_NODES + jnp.arange(pad, dtype=jnp.int32) % (N_ACC - N_NODES)

    def slabs(flat):
        ch = flat.reshape(-1, CH)
        fast = ch[:NS * NCH_F].reshape(NS, NCH_F, CH)
        slow = ch[NS * NCH_F:].reshape(NS, NCH_S, CH)
        slow = jnp.concatenate(
            [slow, jnp.zeros((NS, NCH_F - NCH_S, CH), jnp.int32)], axis=1)
        return jnp.stack([fast, slow], axis=1).reshape(NW, NCH_F, CH)

    src_p = slabs(jnp.concatenate([src, jnp.zeros((pad,), jnp.int32)]))
    dst_p = slabs(jnp.concatenate([dst, pad_dst]))
    z64 = jnp.zeros((N_ACC, EMB), jnp.float32)
    z16 = jnp.zeros((N_ACC, 16), jnp.float32)
    ones = jnp.ones((CH, 16), jnp.float32)

    keep = jax.random.bernoulli(jax.random.key(42), 0.5, (N_NODES, EMB))
    mask2 = keep.astype(jnp.float32) * 2.0

    p1, xr = _tc_pre(node_x, w1l, w1r)
    agg1, cnt = _edge_agg_cnt(p1, src_p, dst_p, z64, z16, ones)
    renc = _tc_renc(rating_mat, w_enc)
    bias = (b2l + b_enc).reshape(1, EMB)
    p2, hrb, csum = _tc_mid(agg1, cnt, xr, mask2, renc,
                            b1l.reshape(1, EMB), bias, w2l, w2r)
    (agg2,) = _edge_agg(p2, src_p, dst_p, z64, z16, ones)
    a0x, a1x, hx, cx = _gather4(agg2, hrb, csum, x)
    return _tc_dec(a0x, a1x, hx, cx, w_dec, b_dec.reshape(1, M_ITEMS))
